# baseline (speedup 1.0000x reference)
import functools
import jax
import jax.numpy as jnp
from jax import lax
from jax.experimental import pallas as pl
from jax.experimental.pallas import tpu as pltpu
from jax.experimental.pallas import tpu_sc as plsc

_V = 1000
_HI = 5.0
_LO = -5.0
_ROWS = 32768
_CH = 64
_NC, _NS, _L = 2, 16, 16
_NW = _NC * _NS
_RPW = _ROWS // _NW
_NCH = _RPW // _CH


def _make_sc_onehot():
    @functools.partial(
        pl.kernel,
        mesh=plsc.VectorSubcoreMesh(core_axis_name="c", subcore_axis_name="s"),
        out_type=jax.ShapeDtypeStruct((_ROWS, _V), jnp.float32),
        scratch_types=[
            pltpu.VMEM((_RPW,), jnp.int32),
            pltpu.VMEM((_CH, _V), jnp.float32),
        ],
        compiler_params=pltpu.CompilerParams(use_tc_tiling_on_sc=False,
                                             needs_layout_passes=False),
    )
    def _sc_onehot(ids_hbm, out_hbm, ids_v, buf):
        wid = lax.axis_index("s") * _NC + lax.axis_index("c")
        base = wid * _RPW
        pltpu.sync_copy(ids_hbm.at[pl.ds(base, _RPW)], ids_v)

        lo = jnp.full((_L,), _LO, jnp.float32)
        hi = jnp.full((_L,), _HI, jnp.float32)
        iota = lax.broadcasted_iota(jnp.int32, (_L,), 0)

        def fill_row(r, carry):
            for c in range(_V // _L):
                buf[r, pl.ds(c * _L, _L)] = lo
            buf[r, pl.ds(_V - _L, _L)] = lo
            return carry

        lax.fori_loop(0, _CH, fill_row, 0)

        def poke(row0, val):
            for g in range(_CH // _L):
                ids16 = ids_v[pl.ds(row0 + g * _L, _L)]
                rows16 = g * _L + iota
                plsc.store_scatter(buf, [rows16, ids16], val)

        def chunk_body(k, carry):
            row0 = k * _CH
            poke(row0, hi)
            pltpu.sync_copy(buf, out_hbm.at[pl.ds(base + row0, _CH)])
            poke(row0, lo)
            return carry

        lax.fori_loop(0, _NCH, chunk_body, 0)

    return _sc_onehot


def kernel(input_ids):
    Bx, Tx = input_ids.shape
    ids = (input_ids.astype(jnp.int32) % _V).reshape(-1)
    out = _make_sc_onehot()(ids)
    return out.reshape(Bx, Tx, _V)


# speedup vs baseline: 2.9304x; 2.9304x over previous
import functools
import jax
import jax.numpy as jnp
from jax import lax
from jax.experimental import pallas as pl
from jax.experimental.pallas import tpu as pltpu
from jax.experimental.pallas import tpu_sc as plsc

_B, _T, _V = 16, 2048, 1000
_HI = 5.0
_LO = -5.0
_NC, _NS, _L = 2, 16, 16
_NW = _NC * _NS
_VCH = 56
_V0SPLIT = 496


def _make_sc_onehot():
    @functools.partial(
        pl.kernel,
        mesh=plsc.VectorSubcoreMesh(core_axis_name="c", subcore_axis_name="s"),
        out_type=jax.ShapeDtypeStruct((_B, _V, _T), jnp.float32),
        scratch_types=[
            pltpu.VMEM((_T,), jnp.int32),
            pltpu.VMEM((_VCH, _T), jnp.float32),
        ],
        compiler_params=pltpu.CompilerParams(use_tc_tiling_on_sc=True,
                                             needs_layout_passes=False),
    )
    def _sc_onehot(ids_hbm, out_hbm, ids_v, buf):
        wid = lax.axis_index("s") * _NC + lax.axis_index("c")
        b = wid // 2
        half = wid % 2
        vbase = half * _V0SPLIT
        pltpu.sync_copy(ids_hbm.at[pl.ds(b * _T, _T)], ids_v)

        lo = jnp.full((_L,), _LO, jnp.float32)
        hi = jnp.full((_L,), _HI, jnp.float32)
        iota = lax.broadcasted_iota(jnp.int32, (_L,), 0)

        def fill_row(r, carry):
            for c in range(_T // _L):
                buf[r, pl.ds(c * _L, _L)] = lo
            return carry

        lax.fori_loop(0, _VCH, fill_row, 0)

        def poke(v0, nv, val):
            for g in range(_T // _L):
                ids16 = ids_v[pl.ds(g * _L, _L)]
                rows16 = ids16 - v0
                cols16 = g * _L + iota
                mask = (ids16 >= v0) & (ids16 < v0 + nv)
                plsc.store_scatter(buf, [rows16, cols16], val, mask=mask)

        def chunk(v0, nv):
            poke(v0, nv, hi)
            pltpu.sync_copy(buf.at[pl.ds(0, nv)],
                            out_hbm.at[b, pl.ds(v0, nv), :])
            poke(v0, nv, lo)

        def chunk_body(k, carry):
            chunk(vbase + k * _VCH, _VCH)
            return carry

        lax.fori_loop(0, 8, chunk_body, 0)

        @pl.when(half == 0)
        def _():
            chunk(vbase + 8 * _VCH, _V0SPLIT - 8 * _VCH)

        @pl.when(half == 1)
        def _():
            chunk(vbase + 8 * _VCH, _VCH)

    return _sc_onehot


def kernel(input_ids):
    Bx, Tx = input_ids.shape
    ids = (input_ids.astype(jnp.int32) % _V).reshape(-1)
    out = _make_sc_onehot()(ids)
    return jnp.swapaxes(out, 1, 2)


# speedup vs baseline: 4.0195x; 1.3717x over previous
import functools
import jax
import jax.numpy as jnp
from jax import lax
from jax.experimental import pallas as pl
from jax.experimental.pallas import tpu as pltpu
from jax.experimental.pallas import tpu_sc as plsc

_B, _T, _V = 16, 2048, 1000
_HI = 5.0
_LO = -5.0
_NC, _NS, _L = 2, 16, 16
_VCH = 24
_V0SPLIT = 496
_NCH = 21


def _make_sc_onehot():
    @functools.partial(
        pl.kernel,
        mesh=plsc.VectorSubcoreMesh(core_axis_name="c", subcore_axis_name="s"),
        out_type=jax.ShapeDtypeStruct((_B, _V, _T), jnp.float32),
        scratch_types=[
            pltpu.VMEM((_T,), jnp.int32),
            pltpu.VMEM((_VCH, _T), jnp.float32),
            pltpu.VMEM((_VCH, _T), jnp.float32),
            pltpu.SemaphoreType.DMA,
            pltpu.SemaphoreType.DMA,
        ],
        compiler_params=pltpu.CompilerParams(use_tc_tiling_on_sc=True,
                                             needs_layout_passes=False),
    )
    def _sc_onehot(ids_hbm, out_hbm, ids_v, buf0, buf1, sem0, sem1):
        wid = lax.axis_index("s") * _NC + lax.axis_index("c")
        b = wid // 2
        half = wid % 2
        vbase = half * _V0SPLIT
        pltpu.sync_copy(ids_hbm.at[pl.ds(b * _T, _T)], ids_v)

        lo = jnp.full((_L,), _LO, jnp.float32)
        hi = jnp.full((_L,), _HI, jnp.float32)
        iota = lax.broadcasted_iota(jnp.int32, (_L,), 0)

        def fill(buf):
            def fill_row(r, carry):
                for c in range(_T // _L):
                    buf[r, pl.ds(c * _L, _L)] = lo
                return carry
            lax.fori_loop(0, _VCH, fill_row, 0)

        def poke(buf, v0, nv, val):
            def poke_g(g8, carry):
                for u in range(8):
                    off = (g8 * 8 + u) * _L
                    ids16 = ids_v[pl.ds(off, _L)]
                    rows16 = ids16 - v0
                    cols16 = off + iota
                    mask = (ids16 >= v0) & (ids16 < v0 + nv)
                    plsc.store_scatter(buf, [rows16, cols16], val, mask=mask)
                return carry
            lax.fori_loop(0, _T // _L // 8, poke_g, 0)

        def start(buf, sem, k, nv):
            v0 = vbase + k * _VCH
            poke(buf, v0, nv, hi)
            pltpu.async_copy(buf.at[pl.ds(0, nv)],
                             out_hbm.at[b, pl.ds(v0, nv), :], sem)

        def wait(buf, sem, nv):
            pltpu.make_async_copy(buf.at[pl.ds(0, nv)],
                                  out_hbm.at[b, pl.ds(0, nv), :], sem).wait()

        def finish(buf, sem, k, nv):
            wait(buf, sem, nv)
            poke(buf, vbase + k * _VCH, nv, lo)

        fill(buf0)
        start(buf0, sem0, 0, _VCH)
        fill(buf1)
        start(buf1, sem1, 1, _VCH)

        def two_body(j, carry):
            k0 = 2 + 2 * j
            finish(buf0, sem0, k0 - 2, _VCH)
            start(buf0, sem0, k0, _VCH)
            finish(buf1, sem1, k0 - 1, _VCH)
            start(buf1, sem1, k0 + 1, _VCH)
            return carry

        lax.fori_loop(0, (_NCH - 3) // 2, two_body, 0)

        finish(buf0, sem0, 18, _VCH)

        @pl.when(half == 0)
        def _():
            start(buf0, sem0, 20, _V0SPLIT - 20 * _VCH)
            wait(buf0, sem0, _V0SPLIT - 20 * _VCH)

        @pl.when(half == 1)
        def _():
            start(buf0, sem0, 20, _VCH)
            wait(buf0, sem0, _VCH)

        wait(buf1, sem1, _VCH)

    return _sc_onehot


def kernel(input_ids):
    Bx, Tx = input_ids.shape
    ids = (input_ids.astype(jnp.int32) % _V).reshape(-1)
    out = _make_sc_onehot()(ids)
    return jnp.swapaxes(out, 1, 2)


# speedup vs baseline: 4.6160x; 1.1484x over previous
import functools
import jax
import jax.numpy as jnp
from jax import lax
from jax.experimental import pallas as pl
from jax.experimental.pallas import tpu as pltpu
from jax.experimental.pallas import tpu_sc as plsc

_B, _T, _V = 16, 2048, 1000
_HI = 5.0
_LO = -5.0
_NC, _NS, _L = 2, 16, 16
_TW = 1024
_VCH = 56
_NFULL = 17
_VREM = _V - _NFULL * _VCH


def _make_sc_onehot():
    @functools.partial(
        pl.kernel,
        mesh=plsc.VectorSubcoreMesh(core_axis_name="c", subcore_axis_name="s"),
        out_type=jax.ShapeDtypeStruct((_B, _V, _T), jnp.float32),
        scratch_types=[
            pltpu.VMEM((_TW,), jnp.int32),
            pltpu.VMEM((_VCH, _TW), jnp.float32),
            pltpu.VMEM((_VCH, _TW), jnp.float32),
            pltpu.SemaphoreType.DMA,
            pltpu.SemaphoreType.DMA,
        ],
        compiler_params=pltpu.CompilerParams(use_tc_tiling_on_sc=True,
                                             needs_layout_passes=False),
    )
    def _sc_onehot(ids_hbm, out_hbm, ids_v, buf0, buf1, sem0, sem1):
        wid = lax.axis_index("s") * _NC + lax.axis_index("c")
        b = wid // 2
        t0 = (wid % 2) * _TW
        pltpu.sync_copy(ids_hbm.at[pl.ds(wid * _TW, _TW)], ids_v)

        lo = jnp.full((_L,), _LO, jnp.float32)
        hi = jnp.full((_L,), _HI, jnp.float32)
        iota = lax.broadcasted_iota(jnp.int32, (_L,), 0)

        def fill(buf):
            def fill_row(r, carry):
                for c in range(_TW // _L):
                    buf[r, pl.ds(c * _L, _L)] = lo
                return carry
            lax.fori_loop(0, _VCH, fill_row, 0)

        def poke(buf, v0, nv, val):
            def poke_g(g8, carry):
                for u in range(8):
                    off = (g8 * 8 + u) * _L
                    ids16 = ids_v[pl.ds(off, _L)]
                    rows16 = ids16 - v0
                    cols16 = off + iota
                    mask = rows16.astype(jnp.uint32) < jnp.uint32(nv)
                    plsc.store_scatter(buf, [rows16, cols16], val, mask=mask)
                return carry
            lax.fori_loop(0, _TW // _L // 8, poke_g, 0)

        def poke2(buf, v_old, nv_old, v_new, nv_new):
            def poke_g(g8, carry):
                for u in range(8):
                    off = (g8 * 8 + u) * _L
                    ids16 = ids_v[pl.ds(off, _L)]
                    cols16 = off + iota
                    r_old = ids16 - v_old
                    m_old = r_old.astype(jnp.uint32) < jnp.uint32(nv_old)
                    plsc.store_scatter(buf, [r_old, cols16], lo, mask=m_old)
                    r_new = ids16 - v_new
                    m_new = r_new.astype(jnp.uint32) < jnp.uint32(nv_new)
                    plsc.store_scatter(buf, [r_new, cols16], hi, mask=m_new)
                return carry
            lax.fori_loop(0, _TW // _L // 8, poke_g, 0)

        def start(buf, sem, k, nv):
            v0 = k * _VCH
            poke(buf, v0, nv, hi)
            pltpu.async_copy(buf.at[pl.ds(0, nv)],
                             out_hbm.at[b, pl.ds(v0, nv), pl.ds(t0, _TW)], sem)

        def wait(buf, sem, nv):
            pltpu.make_async_copy(buf.at[pl.ds(0, nv)],
                                  out_hbm.at[b, pl.ds(0, nv), pl.ds(t0, _TW)],
                                  sem).wait()

        def finish(buf, sem, k, nv):
            wait(buf, sem, nv)
            poke(buf, k * _VCH, nv, lo)

        fill(buf0)
        start(buf0, sem0, 0, _VCH)
        fill(buf1)
        start(buf1, sem1, 1, _VCH)

        def step(buf, sem, k_old, k_new, nv_new):
            wait(buf, sem, _VCH)
            poke2(buf, k_old * _VCH, _VCH, k_new * _VCH, nv_new)
            pltpu.async_copy(buf.at[pl.ds(0, nv_new)],
                             out_hbm.at[b, pl.ds(k_new * _VCH, nv_new),
                                        pl.ds(t0, _TW)], sem)

        def two_body(j, carry):
            k0 = 2 + 2 * j
            step(buf0, sem0, k0 - 2, k0, _VCH)
            step(buf1, sem1, k0 - 1, k0 + 1, _VCH)
            return carry

        lax.fori_loop(0, 7, two_body, 0)

        step(buf0, sem0, 14, 16, _VCH)
        step(buf1, sem1, 15, 17, _VREM)
        wait(buf0, sem0, _VCH)
        wait(buf1, sem1, _VREM)

    return _sc_onehot


def kernel(input_ids):
    Bx, Tx = input_ids.shape
    ids = (input_ids.astype(jnp.int32) % _V).reshape(-1)
    out = _make_sc_onehot()(ids)
    return jnp.swapaxes(out, 1, 2)
